# baseline (speedup 1.0000x reference)
import functools

import jax
import jax.numpy as jnp
from jax import lax
from jax.experimental import pallas as pl
from jax.experimental.pallas import tpu as pltpu
from jax.experimental.pallas import tpu_sc as plsc

B = 16384
K = 16
NC = 2
NS = 16
NW = NC * NS
BPW = B // NW
CH = 128
NCH = BPW // CH
L = 16


def _body(uid_hbm, iid_hbm, ut_hbm, it_hbm, out_hbm,
          uidx, iidx, urows, irows, outv, sem):
    wid = lax.axis_index("s") * NC + lax.axis_index("c")
    base = wid * BPW

    for j in range(NCH):
        pltpu.sync_copy(uid_hbm.at[pl.ds(base + j * CH, CH)], uidx.at[j])
        pltpu.sync_copy(iid_hbm.at[pl.ds(base + j * CH, CH)], iidx.at[j])

    copies = []
    for j in range(NCH):
        copies.append(pltpu.async_copy(ut_hbm.at[uidx.at[j]], urows.at[j], sem))
        copies.append(pltpu.async_copy(it_hbm.at[iidx.at[j]], irows.at[j], sem))
    for c in copies:
        c.wait()

    lane = lax.iota(jnp.int32, L)

    def group(g, carry):
        j = g // (CH // L)
        r0 = (g % (CH // L)) * L
        row_idx = r0 + lane
        acc = jnp.zeros((L,), jnp.float32)
        for k in range(K):
            col = (lane + k) & (K - 1)
            u = plsc.load_gather(urows.at[j], [row_idx, col])
            v = plsc.load_gather(irows.at[j], [row_idx, col])
            acc = acc + u * v
        outv[pl.ds(g * L, L)] = acc
        return carry

    lax.fori_loop(0, BPW // L, group, 0)

    pltpu.sync_copy(outv, out_hbm.at[pl.ds(base, BPW)])


@jax.jit
def kernel(user_ids, item_ids, user_table, item_table):
    mesh = plsc.VectorSubcoreMesh(core_axis_name="c", subcore_axis_name="s")
    f = functools.partial(
        pl.kernel,
        mesh=mesh,
        compiler_params=pltpu.CompilerParams(
            needs_layout_passes=False, use_tc_tiling_on_sc=False),
        out_type=jax.ShapeDtypeStruct((B,), jnp.float32),
        scratch_types=[
            pltpu.VMEM((NCH, CH), jnp.int32),
            pltpu.VMEM((NCH, CH), jnp.int32),
            pltpu.VMEM((NCH, CH, K), jnp.float32),
            pltpu.VMEM((NCH, CH, K), jnp.float32),
            pltpu.VMEM((BPW,), jnp.float32),
            pltpu.SemaphoreType.DMA,
        ],
    )(_body)
    return f(user_ids.astype(jnp.int32), item_ids.astype(jnp.int32),
             user_table, item_table)


# speedup vs baseline: 6.1859x; 6.1859x over previous
import functools

import jax
import jax.numpy as jnp
from jax import lax
from jax.experimental import pallas as pl
from jax.experimental.pallas import tpu as pltpu
from jax.experimental.pallas import tpu_sc as plsc

B = 16384
K = 16
NC = 2
NS = 16
NW = NC * NS
BPW = B // NW
L = 16
NG = BPW // L


def _body(uid_hbm, iid_hbm, ut_hbm, it_hbm, out_hbm,
          uids_v, iids_v, uslab, islab, outv, sem):
    wid = lax.axis_index("s") * NC + lax.axis_index("c")
    base = wid * BPW

    pltpu.sync_copy(uid_hbm.at[pl.ds(base, BPW)], uids_v)
    pltpu.sync_copy(iid_hbm.at[pl.ds(base, BPW)], iids_v)

    lane = lax.iota(jnp.int32, L)

    def group(g, carry):
        uvec = uids_v[pl.ds(g * L, L)]
        ivec = iids_v[pl.ds(g * L, L)]
        for s in range(L):
            ub = uvec[s]
            ib = ivec[s]
            uoff = pl.multiple_of((ub >> 7) * 128, 128)
            ioff = pl.multiple_of((ib >> 7) * 128, 128)
            pltpu.async_copy(ut_hbm.at[:, pl.ds(uoff, 128)], uslab.at[s], sem)
            pltpu.async_copy(it_hbm.at[:, pl.ds(ioff, 128)], islab.at[s], sem)
        pltpu.make_async_copy(ut_hbm.at[:, pl.ds(0, L * 128)], uslab, sem).wait()
        pltpu.make_async_copy(it_hbm.at[:, pl.ds(0, L * 128)], islab, sem).wait()

        ul = uids_v[pl.ds(g * L, L)] & 127
        il = iids_v[pl.ds(g * L, L)] & 127
        acc = jnp.zeros((L,), jnp.float32)
        for kk in range(K):
            kv = (lane + kk) & (K - 1)
            u = plsc.load_gather(uslab, [lane, kv, ul])
            v = plsc.load_gather(islab, [lane, kv, il])
            acc = acc + u * v
        outv[pl.ds(g * L, L)] = acc
        return carry

    lax.fori_loop(0, NG, group, 0)

    pltpu.sync_copy(outv, out_hbm.at[pl.ds(base, BPW)])


@jax.jit
def kernel(user_ids, item_ids, user_table, item_table):
    mesh = plsc.VectorSubcoreMesh(core_axis_name="c", subcore_axis_name="s")
    f = functools.partial(
        pl.kernel,
        mesh=mesh,
        compiler_params=pltpu.CompilerParams(needs_layout_passes=False),
        out_type=jax.ShapeDtypeStruct((B,), jnp.float32),
        scratch_types=[
            pltpu.VMEM((BPW,), jnp.int32),
            pltpu.VMEM((BPW,), jnp.int32),
            pltpu.VMEM((L, K, 128), jnp.float32),
            pltpu.VMEM((L, K, 128), jnp.float32),
            pltpu.VMEM((BPW,), jnp.float32),
            pltpu.SemaphoreType.DMA,
        ],
    )(_body)
    return f(user_ids.astype(jnp.int32), item_ids.astype(jnp.int32),
             user_table.T, item_table.T)
